# baseline (speedup 1.0000x reference)
import functools

import jax
import jax.numpy as jnp
from jax import lax
from jax.experimental import pallas as pl
from jax.experimental.pallas import tpu as pltpu
from jax.experimental.pallas import tpu_sc as plsc

N = 100000
M = 1000
D = 128
E = 600000

L = 16
NC = 2
NS = 16
NW = NC * NS

DP = D // 2

EPAD = 600064
PER_W = EPAD // NW
CHUNK = 64
NCHUNK = PER_W // CHUNK
IDX_ROWS = EPAD // CHUNK


def _sqrt16(x):
    xe = jnp.maximum(x, jnp.float32(1e-30))
    i = plsc.bitcast(xe, jnp.int32)
    yi = jnp.int32(0x5F3759DF) - (i >> 1)
    y = plsc.bitcast(yi, jnp.float32)
    half = jnp.float32(0.5) * xe
    for _ in range(3):
        y = y * (jnp.float32(1.5) - half * y * y)
    return xe * y


def _sc_body(h_hbm, g_hbm, hidx_hbm, tidx_hbm, ridx_hbm, out_hbm,
             hidx_v, tidx_v, ridx_v,
             rows_sa, rows_ta, rows_ra, rows_sb, rows_tb, rows_rb, out_v,
             sem_sa, sem_ta, sem_ra, sem_sb, sem_tb, sem_rb):
    wid = lax.axis_index("s") * NC + lax.axis_index("c")
    base = wid * PER_W

    pltpu.sync_copy(hidx_hbm.at[pl.ds(base, PER_W)], hidx_v)
    pltpu.sync_copy(tidx_hbm.at[pl.ds(base, PER_W)], tidx_v)
    pltpu.sync_copy(ridx_hbm.at[pl.ds(base, PER_W)], ridx_v)

    lane = lax.broadcasted_iota(jnp.int32, (L,), 0)

    def start(c, rows_s, rows_t, rows_r, sem_s, sem_t, sem_r):
        pltpu.async_copy(h_hbm.at[hidx_v.at[pl.ds(c * CHUNK, CHUNK)]],
                         rows_s, sem_s)
        pltpu.async_copy(h_hbm.at[tidx_v.at[pl.ds(c * CHUNK, CHUNK)]],
                         rows_t, sem_t)
        pltpu.async_copy(g_hbm.at[ridx_v.at[pl.ds(c * CHUNK, CHUNK)]],
                         rows_r, sem_r)

    def wait(c, rows_s, rows_t, rows_r, sem_s, sem_t, sem_r):
        idx0 = pl.ds(c * CHUNK, CHUNK)
        pltpu.make_async_copy(h_hbm.at[hidx_v.at[idx0]], rows_s, sem_s).wait()
        pltpu.make_async_copy(h_hbm.at[tidx_v.at[idx0]], rows_t, sem_t).wait()
        pltpu.make_async_copy(g_hbm.at[ridx_v.at[idx0]], rows_r, sem_r).wait()

    def compute(c, rows_s, rows_t, rows_r):
        def group_step(g4, carry2):
            row = g4 * L + lane

            UNROLL = 16

            def dim_step(kk, accs):
                acc_hi, acc_lo = accs
                for k2 in range(UNROLL):
                    col = (kk * UNROLL + k2) ^ lane
                    ws = plsc.load_gather(rows_s, [row, col])
                    wt = plsc.load_gather(rows_t, [row, col])
                    wr = plsc.load_gather(rows_r, [row, col])
                    s16 = plsc.bitcast(ws, jnp.bfloat16)
                    t16 = plsc.bitcast(wt, jnp.bfloat16)
                    r16 = plsc.bitcast(wr, jnp.bfloat16)
                    d16 = (s16 + r16) - t16
                    wd = plsc.bitcast(d16, jnp.int32)
                    dhi = plsc.bitcast(wd & jnp.int32(-65536), jnp.float32)
                    dlo = plsc.bitcast(wd << 16, jnp.float32)
                    acc_hi = acc_hi + dhi * dhi
                    acc_lo = acc_lo + dlo * dlo
                return acc_hi, acc_lo

            zero = jnp.zeros((L,), jnp.float32)
            acc_hi, acc_lo = lax.fori_loop(0, DP // UNROLL, dim_step,
                                           (zero, zero))
            out_v[pl.ds(c * CHUNK + g4 * L, L)] = _sqrt16(acc_hi + acc_lo)
            return carry2

        lax.fori_loop(0, CHUNK // L, group_step, jnp.int32(0))

    bufs_a = (rows_sa, rows_ta, rows_ra, sem_sa, sem_ta, sem_ra)
    bufs_b = (rows_sb, rows_tb, rows_rb, sem_sb, sem_tb, sem_rb)

    start(0, *bufs_a)

    def pair_step(p, carry):
        c0 = p * 2
        start(c0 + 1, *bufs_b)
        wait(c0, *bufs_a)
        compute(c0, *bufs_a[:3])
        start(c0 + 2, *bufs_a)
        wait(c0 + 1, *bufs_b)
        compute(c0 + 1, *bufs_b[:3])
        return carry

    lax.fori_loop(0, (NCHUNK - 1) // 2, pair_step, jnp.int32(0))
    wait(NCHUNK - 1, *bufs_a)
    compute(NCHUNK - 1, *bufs_a[:3])

    pltpu.sync_copy(out_v, out_hbm.at[pl.ds(wid * PER_W, PER_W)])


@jax.jit
def _sc_scores(h, g, hidx, tidx, ridx):
    mesh = plsc.VectorSubcoreMesh(core_axis_name="c", subcore_axis_name="s")
    f = pl.kernel(
        _sc_body,
        out_type=jax.ShapeDtypeStruct((EPAD,), jnp.float32),
        mesh=mesh,
        compiler_params=pltpu.CompilerParams(needs_layout_passes=False,
                                             use_tc_tiling_on_sc=False),
        scratch_types=[
            pltpu.VMEM((PER_W,), jnp.int32),
            pltpu.VMEM((PER_W,), jnp.int32),
            pltpu.VMEM((PER_W,), jnp.int32),
            pltpu.VMEM((CHUNK, DP), jnp.int32),
            pltpu.VMEM((CHUNK, DP), jnp.int32),
            pltpu.VMEM((CHUNK, DP), jnp.int32),
            pltpu.VMEM((CHUNK, DP), jnp.int32),
            pltpu.VMEM((CHUNK, DP), jnp.int32),
            pltpu.VMEM((CHUNK, DP), jnp.int32),
            pltpu.VMEM((PER_W,), jnp.float32),
            pltpu.SemaphoreType.DMA,
            pltpu.SemaphoreType.DMA,
            pltpu.SemaphoreType.DMA,
            pltpu.SemaphoreType.DMA,
            pltpu.SemaphoreType.DMA,
            pltpu.SemaphoreType.DMA,
        ],
    )
    return f(h, g, hidx, tidx, ridx)


def _pack_bf16(x):
    n = x.shape[0]
    return jax.lax.bitcast_convert_type(
        x.astype(jnp.bfloat16).reshape(n, DP, 2), jnp.int32)


def kernel(h, g, eval_idx, eval_type):
    pad = EPAD - E
    head = jnp.pad(eval_idx[0].astype(jnp.int32), (0, pad))
    tail = jnp.pad(eval_idx[1].astype(jnp.int32), (0, pad))
    rel = jnp.pad(eval_type.astype(jnp.int32), (0, pad))
    scores = _sc_scores(_pack_bf16(h), _pack_bf16(g), head, tail, rel)
    return scores[:E]


# speedup vs baseline: 1.9685x; 1.9685x over previous
import functools

import jax
import jax.numpy as jnp
from jax import lax
from jax.experimental import pallas as pl
from jax.experimental.pallas import tpu as pltpu
from jax.experimental.pallas import tpu_sc as plsc

N = 100000
M = 1000
D = 128
E = 600000

L = 16
NC = 2
NS = 16
NW = NC * NS

DP = D // 2

EPAD = 600064
PER_W = EPAD // NW
CHUNK = 64
NCHUNK = PER_W // CHUNK
IDX_ROWS = EPAD // CHUNK


def _sqrt16(x):
    xe = jnp.maximum(x, jnp.float32(1e-30))
    i = plsc.bitcast(xe, jnp.int32)
    yi = jnp.int32(0x5F3759DF) - (i >> 1)
    y = plsc.bitcast(yi, jnp.float32)
    half = jnp.float32(0.5) * xe
    for _ in range(3):
        y = y * (jnp.float32(1.5) - half * y * y)
    return xe * y


def _sc_body(h_hbm, g_hbm, hidx_hbm, tidx_hbm, ridx_hbm, out_hbm,
             hidx_v, tidx_v, ridx_v,
             rows_sa, rows_ta, rows_ra, rows_sb, rows_tb, rows_rb, out_v,
             sem_sa, sem_ta, sem_ra, sem_sb, sem_tb, sem_rb):
    wid = lax.axis_index("s") * NC + lax.axis_index("c")
    base = wid * PER_W

    pltpu.sync_copy(hidx_hbm.at[pl.ds(base, PER_W)], hidx_v)
    pltpu.sync_copy(tidx_hbm.at[pl.ds(base, PER_W)], tidx_v)
    pltpu.sync_copy(ridx_hbm.at[pl.ds(base, PER_W)], ridx_v)

    lane = lax.broadcasted_iota(jnp.int32, (L,), 0)

    def start(c, rows_s, rows_t, rows_r, sem_s, sem_t, sem_r):
        pltpu.async_copy(h_hbm.at[hidx_v.at[pl.ds(c * CHUNK, CHUNK)]],
                         rows_s, sem_s)
        pltpu.async_copy(h_hbm.at[tidx_v.at[pl.ds(c * CHUNK, CHUNK)]],
                         rows_t, sem_t)
        pltpu.async_copy(g_hbm.at[ridx_v.at[pl.ds(c * CHUNK, CHUNK)]],
                         rows_r, sem_r)

    def wait(c, rows_s, rows_t, rows_r, sem_s, sem_t, sem_r):
        idx0 = pl.ds(c * CHUNK, CHUNK)
        pltpu.make_async_copy(h_hbm.at[hidx_v.at[idx0]], rows_s, sem_s).wait()
        pltpu.make_async_copy(h_hbm.at[tidx_v.at[idx0]], rows_t, sem_t).wait()
        pltpu.make_async_copy(g_hbm.at[ridx_v.at[idx0]], rows_r, sem_r).wait()

    def compute(c, rows_s, rows_t, rows_r):
        def group_step(g4, carry2):
            row = g4 * L + lane

            UNROLL = 16

            def dim_step(kk, accs):
                acc_hi, acc_lo = accs
                for k2 in range(UNROLL):
                    col = (kk * UNROLL + k2) ^ lane
                    ws = plsc.load_gather(rows_s, [row, col])
                    wt = plsc.load_gather(rows_t, [row, col])
                    wr = plsc.load_gather(rows_r, [row, col])
                    s16 = plsc.bitcast(ws, jnp.bfloat16)
                    t16 = plsc.bitcast(wt, jnp.bfloat16)
                    r16 = plsc.bitcast(wr, jnp.bfloat16)
                    d16 = (s16 + r16) - t16
                    wd = plsc.bitcast(d16, jnp.int32)
                    dhi = plsc.bitcast(wd & jnp.int32(-65536), jnp.float32)
                    dlo = plsc.bitcast(wd << 16, jnp.float32)
                    acc_hi = acc_hi + dhi * dhi
                    acc_lo = acc_lo + dlo * dlo
                return acc_hi, acc_lo

            zero = jnp.zeros((L,), jnp.float32)
            acc_hi, acc_lo = lax.fori_loop(0, DP // UNROLL, dim_step,
                                           (zero, zero))
            out_v[pl.ds(c * CHUNK + g4 * L, L)] = _sqrt16(acc_hi + acc_lo)
            return carry2

        lax.fori_loop(0, CHUNK // L, group_step, jnp.int32(0))

    bufs_a = (rows_sa, rows_ta, rows_ra, sem_sa, sem_ta, sem_ra)
    bufs_b = (rows_sb, rows_tb, rows_rb, sem_sb, sem_tb, sem_rb)

    start(0, *bufs_a)

    def pair_step(p, carry):
        c0 = p * 2
        start(c0 + 1, *bufs_b)
        wait(c0, *bufs_a)
        compute(c0, *bufs_a[:3])
        start(c0 + 2, *bufs_a)
        wait(c0 + 1, *bufs_b)
        compute(c0 + 1, *bufs_b[:3])
        return carry

    lax.fori_loop(0, (NCHUNK - 1) // 2, pair_step, jnp.int32(0))
    wait(NCHUNK - 1, *bufs_a)
    compute(NCHUNK - 1, *bufs_a[:3])

    pltpu.sync_copy(out_v, out_hbm.at[pl.ds(wid * PER_W, PER_W)])


@jax.jit
def _sc_scores(h, g, hidx, tidx, ridx):
    mesh = plsc.VectorSubcoreMesh(core_axis_name="c", subcore_axis_name="s")
    f = pl.kernel(
        _sc_body,
        out_type=jax.ShapeDtypeStruct((EPAD,), jnp.float32),
        mesh=mesh,
        compiler_params=pltpu.CompilerParams(needs_layout_passes=False,
                                             use_tc_tiling_on_sc=False),
        scratch_types=[
            pltpu.VMEM((PER_W,), jnp.int32),
            pltpu.VMEM((PER_W,), jnp.int32),
            pltpu.VMEM((PER_W,), jnp.int32),
            pltpu.VMEM((CHUNK, DP), jnp.int32),
            pltpu.VMEM((CHUNK, DP), jnp.int32),
            pltpu.VMEM((CHUNK, DP), jnp.int32),
            pltpu.VMEM((CHUNK, DP), jnp.int32),
            pltpu.VMEM((CHUNK, DP), jnp.int32),
            pltpu.VMEM((CHUNK, DP), jnp.int32),
            pltpu.VMEM((PER_W,), jnp.float32),
            pltpu.SemaphoreType.DMA,
            pltpu.SemaphoreType.DMA,
            pltpu.SemaphoreType.DMA,
            pltpu.SemaphoreType.DMA,
            pltpu.SemaphoreType.DMA,
            pltpu.SemaphoreType.DMA,
        ],
    )
    return f(h, g, hidx, tidx, ridx)


def _pack_bf16(x):
    lo = jax.lax.bitcast_convert_type(x[:, :DP].astype(jnp.bfloat16), jnp.int16)
    hi = jax.lax.bitcast_convert_type(x[:, DP:].astype(jnp.bfloat16), jnp.int16)
    return ((hi.astype(jnp.int32) << 16)
            | (lo.astype(jnp.int32) & jnp.int32(0xFFFF)))


def kernel(h, g, eval_idx, eval_type):
    pad = EPAD - E
    head = jnp.pad(eval_idx[0].astype(jnp.int32), (0, pad))
    tail = jnp.pad(eval_idx[1].astype(jnp.int32), (0, pad))
    rel = jnp.pad(eval_type.astype(jnp.int32), (0, pad))
    scores = _sc_scores(_pack_bf16(h), _pack_bf16(g), head, tail, rel)
    return scores[:E]
